# baseline (speedup 1.0000x reference)
import jax
import jax.numpy as jnp
from jax import lax
from jax.experimental import pallas as pl
from jax.experimental.pallas import tpu as pltpu
from jax.experimental.pallas import tpu_sc as plsc

EMBED_DIM = 64
BATCH = 16384
N_FIELDS = 26
NUM_FEAT = 1000000

NUM_CORES = 2
NUM_SUBCORES = 16
NUM_WORKERS = NUM_CORES * NUM_SUBCORES

CB = 128
N_BLOCKS = N_FIELDS * (BATCH // CB)
BLOCKS_PER_WORKER = N_BLOCKS // NUM_WORKERS
TCOLS = BATCH // CB


def _gather_body(wp_hbm, idx_hbm, out_hbm,
                 idx_v, pidx_v, col_v, rows_v, out_v, sem):
    wid = lax.axis_index("s") * NUM_CORES + lax.axis_index("c")
    iota = lax.iota(jnp.int32, 16)

    def block_step(t, carry):
        k = wid + NUM_WORKERS * t
        f = lax.shift_right_logical(k, 7)
        tc = lax.bitwise_and(k, TCOLS - 1)

        pltpu.sync_copy(idx_hbm.at[k], idx_v)

        def prep(g, c2):
            v = idx_v[pl.ds(16 * g, 16)]
            pidx_v[pl.ds(16 * g, 16)] = lax.shift_right_logical(v, 1)
            col_v[pl.ds(16 * g, 16)] = lax.shift_left(
                lax.bitwise_and(v, 1), 6)
            return c2

        lax.fori_loop(0, CB // 16, prep, 0)

        pltpu.async_copy(wp_hbm.at[pidx_v], rows_v, sem).wait()

        def grp(g, c2):
            c_vec = 16 * g + iota
            base_col = col_v[pl.ds(16 * g, 16)]
            for j in range(EMBED_DIM):
                vals = plsc.load_gather(rows_v, [c_vec, base_col + j])
                out_v[j // 8, j % 8, pl.ds(16 * g, 16)] = vals
            return c2

        lax.fori_loop(0, CB // 16, grp, 0)

        pltpu.sync_copy(out_v, out_hbm.at[f, :, tc])
        return carry

    lax.fori_loop(0, BLOCKS_PER_WORKER, block_step, 0)


def kernel(input_indices, W):
    w_pairs = W.reshape(NUM_FEAT // 2, 2 * EMBED_DIM)
    idx2d = input_indices.T.astype(jnp.int32).reshape(N_BLOCKS, CB)
    mesh = plsc.VectorSubcoreMesh(core_axis_name="c", subcore_axis_name="s")

    out5d = pl.kernel(
        _gather_body,
        out_type=jax.ShapeDtypeStruct(
            (N_FIELDS, EMBED_DIM // 8, TCOLS, 8, CB), jnp.float32),
        mesh=mesh,
        scratch_types=[
            pltpu.VMEM((CB,), jnp.int32),
            pltpu.VMEM((CB,), jnp.int32),
            pltpu.VMEM((CB,), jnp.int32),
            pltpu.VMEM((CB, 2 * EMBED_DIM), jnp.float32),
            pltpu.VMEM((EMBED_DIM // 8, 8, CB), jnp.float32),
            pltpu.SemaphoreType.DMA,
        ],
        compiler_params=pltpu.CompilerParams(needs_layout_passes=False),
    )(w_pairs, idx2d)

    return out5d.transpose((2, 4, 0, 1, 3)).reshape(BATCH, N_FIELDS, EMBED_DIM)
